# baseline (speedup 1.0000x reference)
import functools

import jax
import jax.numpy as jnp
from jax import lax
from jax.experimental import pallas as pl
from jax.experimental.pallas import tpu as pltpu
from jax.experimental.pallas import tpu_sc as plsc

def _att_sc_body(nnode, objrow_hbm, att0_hbm, out_hbm, idxv, rowsv, sem):
    info = plsc.get_sparse_core_info()
    wid = lax.axis_index("s") * info.num_cores + lax.axis_index("c")

    @pl.when(wid == 0)
    def _():
        pltpu.sync_copy(objrow_hbm, idxv)
        idx = idxv[0, :]
        pltpu.async_copy(att0_hbm.at[idx], rowsv, sem).wait()
        pltpu.sync_copy(rowsv, out_hbm)


def _gather_att_sc(objrow, att0, nnode):
    feat = att0.shape[-1]
    mesh = plsc.VectorSubcoreMesh(core_axis_name="c", subcore_axis_name="s")
    body = functools.partial(_att_sc_body, nnode)
    fn = pl.kernel(
        body,
        mesh=mesh,
        out_type=jax.ShapeDtypeStruct((16, feat), jnp.float32),
        scratch_types=[
            pltpu.VMEM((1, 16), jnp.int32),
            pltpu.VMEM((16, feat), jnp.float32),
            pltpu.SemaphoreType.DMA,
        ],
    )
    return fn(objrow, att0)


def _gpn_core(nbatch, seg, nnode, nobj,
              obj_ref, pool_ref, a_ref, w1_ref, b1_ref, w2_ref,
              b2_ref, p1_ref, pb1_ref, p2_ref, pb2_ref, masks_ref,
              loss_ref, score_ref, objrow_ref, fc_ref, msk_ref, keep_ref):
    f32 = jnp.float32
    gb = 2 * nbatch * seg
    rows = 2 * seg

    lane = jax.lax.broadcasted_iota(jnp.int32, (rows, nobj), 1)
    z_rows = []
    for bb in range(nbatch):
        obj_b = obj_ref[bb * rows:(bb + 1) * rows, :]
        pool_b = pool_ref[bb * rows:(bb + 1) * rows, :]
        wp = jnp.zeros((rows, nobj), f32)
        for n in range(nnode):
            wp = wp + jnp.where(lane == obj_b[:, n:n + 1],
                                pool_b[:, n:n + 1], 0.0)
        aw = jax.lax.dot_general(a_ref[bb], w1_ref[...],
                                 (((1,), (1,)), ((), ())),
                                 preferred_element_type=f32)
        h = jnp.dot(wp, aw, preferred_element_type=f32) + b1_ref[...]
        h = jnp.maximum(h, 0.0)
        z_rows.append(jax.lax.dot_general(w2_ref[...], h,
                                          (((1,), (1,)), ((), ())),
                                          preferred_element_type=f32))
    z = jnp.concatenate(z_rows, axis=0)
    score = jax.nn.sigmoid(z + b2_ref[0, 0])

    col = jax.lax.broadcasted_iota(jnp.int32, (nbatch, rows), 1)
    logp = jnp.maximum(jnp.log(score), -100.0)
    log1m = jnp.maximum(jnp.log(1.0 - score), -100.0)
    contrib = jnp.where(col < seg, logp, log1m)
    loss_ref[0] = -jnp.sum(contrib) / gb

    s400 = score[0:1]
    i400 = jax.lax.broadcasted_iota(jnp.int32, (1, rows), 1)
    m = jnp.max(s400)
    r = jnp.max(jnp.where(s400 == m, i400, -1))
    keep_ref[0] = r
    score_ref[0] = m

    pool_row = pool_ref[pl.ds(r, 1), :]
    obj_row = obj_ref[pl.ds(r, 1), :]
    msk_ref[...] = masks_ref[pl.ds(r, 1), :]
    objrow_ref[...] = obj_row
    acc = jnp.zeros((1, a_ref.shape[2]), f32)
    for n in range(nnode):
        o = obj_row[0, n]
        rown = a_ref[0, pl.ds(o, 1), :]
        acc = acc + pool_row[0, n] * rown
    fc1 = jax.lax.dot_general(acc, p1_ref[...], (((1,), (1,)), ((), ())),
                              preferred_element_type=f32) + pb1_ref[...]
    fc2 = jax.lax.dot_general(fc1, p2_ref[...], (((1,), (1,)), ((), ())),
                              preferred_element_type=f32) + pb2_ref[...]
    fc_ref[...] = fc2


def kernel(b, N, K, L, gpn_obj_ind, gpn_pred_ind, gpn_nrel_ind, gpn_pool_mtx,
           att_feats, x_pred, fc_feats, att_masks, W1, b1, W2, b2, P1, pb1,
           P2, pb2):
    nbatch, _, seg, nnode = gpn_obj_ind.shape
    nobj = att_feats.shape[1]
    feat = att_feats.shape[2]
    hid = W1.shape[0]
    gb = 2 * nbatch * seg

    obj2 = gpn_obj_ind.reshape(gb, nnode).astype(jnp.int32)
    obj2 = jnp.pad(obj2, ((0, 0), (0, 16 - nnode)))
    pool2 = gpn_pool_mtx.reshape(gb, nnode)
    masks0 = att_masks.reshape(gb, nnode)

    core = functools.partial(_gpn_core, nbatch, seg, nnode, nobj)
    outs = pl.pallas_call(
        core,
        out_shape=[
            jax.ShapeDtypeStruct((1,), jnp.float32),
            jax.ShapeDtypeStruct((1,), jnp.float32),
            jax.ShapeDtypeStruct((1, 16), jnp.int32),
            jax.ShapeDtypeStruct((1, feat), jnp.float32),
            jax.ShapeDtypeStruct((1, nnode), jnp.float32),
            jax.ShapeDtypeStruct((1,), jnp.int32),
        ],
        out_specs=[
            pl.BlockSpec(memory_space=pltpu.SMEM),
            pl.BlockSpec(memory_space=pltpu.SMEM),
            pl.BlockSpec(),
            pl.BlockSpec(),
            pl.BlockSpec(),
            pl.BlockSpec(memory_space=pltpu.SMEM),
        ],
    )(obj2, pool2, att_feats, W1, b1.reshape(1, hid), W2,
      b2.reshape(1, 1), P1, pb1.reshape(1, hid), P2, pb2.reshape(1, feat),
      masks0)

    o_loss, o_score, o_objrow, o_fc, o_msk, o_keep = outs
    att = _gather_att_sc(o_objrow, att_feats[0], nnode)[0:nnode]
    return (o_loss.reshape(()), o_score, att[None], o_fc, o_msk, o_keep)


# speedup vs baseline: 1.9058x; 1.9058x over previous
import functools

import jax
import jax.numpy as jnp
from jax.experimental import pallas as pl
from jax.experimental.pallas import tpu as pltpu


def _gpn_core(nbatch, seg, nnode, nobj,
              obj_ref, pool_ref, a_ref, w1_ref, b1_ref, w2_ref, b2_ref,
              p1_ref, pb1_ref, p2_ref, pb2_ref, masks_ref,
              loss_ref, score_ref, att_ref, fc_ref, msk_ref, keep_ref):
    f32 = jnp.float32
    gb = 2 * nbatch * seg
    rows = 2 * seg

    lane = jax.lax.broadcasted_iota(jnp.int32, (rows, nobj), 1)
    z_rows = []
    for bb in range(nbatch):
        obj_b = obj_ref[bb * rows:(bb + 1) * rows, :]
        pool_b = pool_ref[bb * rows:(bb + 1) * rows, :]
        wp = jnp.zeros((rows, nobj), f32)
        for n in range(nnode):
            wp = wp + jnp.where(lane == obj_b[:, n:n + 1],
                                pool_b[:, n:n + 1], 0.0)
        aw = jax.lax.dot_general(a_ref[bb], w1_ref[...],
                                 (((1,), (1,)), ((), ())),
                                 preferred_element_type=f32)
        h = jnp.dot(wp, aw, preferred_element_type=f32) + b1_ref[...]
        h = jnp.maximum(h, 0.0)
        z_rows.append(jax.lax.dot_general(w2_ref[...], h,
                                          (((1,), (1,)), ((), ())),
                                          preferred_element_type=f32))
    z = jnp.concatenate(z_rows, axis=0)
    score = jax.nn.sigmoid(z + b2_ref[0, 0])

    col = jax.lax.broadcasted_iota(jnp.int32, (nbatch, rows), 1)
    logp = jnp.maximum(jnp.log(score), -100.0)
    log1m = jnp.maximum(jnp.log(1.0 - score), -100.0)
    contrib = jnp.where(col < seg, logp, log1m)
    loss_ref[0] = -jnp.sum(contrib) / gb

    s400 = score[0:1]
    i400 = jax.lax.broadcasted_iota(jnp.int32, (1, rows), 1)
    m = jnp.max(s400)
    r = jnp.max(jnp.where(s400 == m, i400, -1))
    keep_ref[0] = r
    score_ref[0] = m

    pool_row = pool_ref[pl.ds(r, 1), :]
    obj_row = obj_ref[pl.ds(r, 1), :]
    msk_ref[...] = masks_ref[pl.ds(r, 1), :]
    acc = jnp.zeros((1, a_ref.shape[2]), f32)
    for n in range(nnode):
        o = obj_row[0, n]
        rown = a_ref[0, pl.ds(o, 1), :]
        att_ref[0, n:n + 1, :] = rown
        acc = acc + pool_row[0, n] * rown
    fc1 = jax.lax.dot_general(acc, p1_ref[...], (((1,), (1,)), ((), ())),
                              preferred_element_type=f32) + pb1_ref[...]
    fc2 = jax.lax.dot_general(fc1, p2_ref[...], (((1,), (1,)), ((), ())),
                              preferred_element_type=f32) + pb2_ref[...]
    fc_ref[...] = fc2


def kernel(b, N, K, L, gpn_obj_ind, gpn_pred_ind, gpn_nrel_ind, gpn_pool_mtx,
           att_feats, x_pred, fc_feats, att_masks, W1, b1, W2, b2, P1, pb1,
           P2, pb2):
    nbatch, _, seg, nnode = gpn_obj_ind.shape
    nobj = att_feats.shape[1]
    feat = att_feats.shape[2]
    hid = W1.shape[0]
    gb = 2 * nbatch * seg

    obj2 = gpn_obj_ind.reshape(gb, nnode).astype(jnp.int32)
    pool2 = gpn_pool_mtx.reshape(gb, nnode)
    masks0 = att_masks.reshape(gb, nnode)

    core = functools.partial(_gpn_core, nbatch, seg, nnode, nobj)
    outs = pl.pallas_call(
        core,
        out_shape=[
            jax.ShapeDtypeStruct((1,), jnp.float32),
            jax.ShapeDtypeStruct((1,), jnp.float32),
            jax.ShapeDtypeStruct((1, nnode, feat), jnp.float32),
            jax.ShapeDtypeStruct((1, feat), jnp.float32),
            jax.ShapeDtypeStruct((1, nnode), jnp.float32),
            jax.ShapeDtypeStruct((1,), jnp.int32),
        ],
        out_specs=[
            pl.BlockSpec(memory_space=pltpu.SMEM),
            pl.BlockSpec(memory_space=pltpu.SMEM),
            pl.BlockSpec(),
            pl.BlockSpec(),
            pl.BlockSpec(),
            pl.BlockSpec(memory_space=pltpu.SMEM),
        ],
    )(obj2, pool2, att_feats, W1, b1.reshape(1, hid), W2, b2.reshape(1, 1),
      P1, pb1.reshape(1, hid), P2, pb2.reshape(1, feat), masks0)

    o_loss, o_score, o_att, o_fc, o_msk, o_keep = outs
    return (o_loss.reshape(()), o_score, o_att, o_fc, o_msk, o_keep)
